# baseline (speedup 1.0000x reference)
import functools

import jax
import jax.numpy as jnp
from jax import lax
from jax.experimental import pallas as pl
from jax.experimental.pallas import tpu as pltpu
from jax.experimental.pallas import tpu_sc as plsc

_L = 2048
_V = 257
_D = 64
_GROWS = 4424
_SMIN = -(_L + 127)
_SMAX = 128

_NC = 2
_NS = 16
_NW = _NC * _NS
_ROWS_PER_W = _L // _NW
_CHUNK = 512
_NCHUNK = _L // _CHUNK
_SPAN = 584


def _build_g(table):
    def body(tab_ref, g_ref):
        t0 = tab_ref[0:1, :]
        t256 = tab_ref[256:257, :]
        g_ref[0:2048, :] = jnp.broadcast_to(t0, (2048, _D))
        g_ref[2048:2304, :] = tab_ref[1:257, :]
        g_ref[2304:_GROWS, :] = jnp.broadcast_to(t256, (_GROWS - 2304, _D))

    return pl.pallas_call(
        body,
        out_shape=jax.ShapeDtypeStruct((_GROWS, _D), jnp.float32),
    )(table)


@functools.partial(
    pl.kernel,
    mesh=plsc.VectorSubcoreMesh(core_axis_name="c", subcore_axis_name="s"),
    out_type=jax.ShapeDtypeStruct((_L, _L, _D), jnp.float32),
    compiler_params=pltpu.CompilerParams(use_tc_tiling_on_sc=True),
    scratch_types=[
        pltpu.VMEM((_SPAN, _D), jnp.float32),
        pltpu.VMEM((16,), jnp.int32),
        pltpu.SemaphoreType.DMA,
    ],
)
def _sc_expand(g_hbm, delta_hbm, out_hbm, buf, delta_v, sem):
    cid = lax.axis_index("c")
    sid = lax.axis_index("s")

    pltpu.sync_copy(delta_hbm, delta_v)
    delta = delta_v[...][0]

    wid = cid * _NS + sid
    base = wid * _ROWS_PER_W

    def start_of(i):
        return jnp.clip(delta - i, _SMIN, _SMAX) - _SMIN

    smin = start_of(base + _ROWS_PER_W - 1)
    lo0 = pl.multiple_of(smin & ~jnp.int32(7), 8)

    def chunk_pass(c, carry):
        lo = pl.multiple_of(lo0 + c * _CHUNK, 8)
        pltpu.sync_copy(g_hbm.at[pl.ds(lo, _SPAN)], buf)

        def fire(r, cr):
            i = base + r
            off = start_of(i) + c * _CHUNK - lo
            pltpu.make_async_copy(
                buf.at[pl.ds(off, _CHUNK)],
                out_hbm.at[i, pl.ds(c * _CHUNK, _CHUNK)],
                sem).start()
            return cr

        lax.fori_loop(0, _ROWS_PER_W, fire, 0)

        def drain(r, cr):
            pltpu.make_async_copy(
                buf.at[pl.ds(0, _CHUNK)],
                out_hbm.at[base, pl.ds(0, _CHUNK)],
                sem).wait()
            return cr

        lax.fori_loop(0, _ROWS_PER_W, drain, 0)
        return carry

    lax.fori_loop(0, _NCHUNK, chunk_pass, 0)


def kernel(length_q, length_k, embeddings_table):
    delta = (jnp.asarray(length_k, jnp.int32)
             - jnp.asarray(length_q, jnp.int32))
    delta16 = jnp.broadcast_to(delta.reshape(1), (16,))
    g = _build_g(embeddings_table)
    return _sc_expand(g, delta16)


# speedup vs baseline: 1.3298x; 1.3298x over previous
import functools

import jax
import jax.numpy as jnp
from jax import lax
from jax.experimental import pallas as pl
from jax.experimental.pallas import tpu as pltpu
from jax.experimental.pallas import tpu_sc as plsc

_L = 2048
_V = 257
_D = 64
_GPAD = 4608
_PCOLS = 4224
_NPH = 128
_SEG = 384
_SEGW = 512
_NSEG = _PCOLS // _SEG
_SMIN = -(_L + 127)
_SMAX = 128

_NC = 2
_NS = 16
_NW = _NC * _NS
_ROWS_PER_W = _L // _NW
_CHUNK = 512
_NCHUNK = _L // _CHUNK
_NSLAB = _ROWS_PER_W * _NCHUNK


def _build_gt_segs(table):
    def seg_cols(gt_parts, a, b):
        t0, tab_t, t256 = gt_parts
        parts = []
        if a < 2048:
            parts.append(jnp.broadcast_to(t0, (_D, min(b, 2048) - a)))
        if b > 2048 and a < 2304:
            lo, hi = max(a, 2048), min(b, 2304)
            parts.append(tab_t[:, lo - 2047:hi - 2047])
        if b > 2304:
            lo = max(a, 2304)
            parts.append(jnp.broadcast_to(t256, (_D, b - lo)))
        return jnp.concatenate(parts, axis=1) if len(parts) > 1 else parts[0]

    def body(tab_ref, seg_ref):
        eye = jnp.eye(_V, dtype=jnp.float32)
        tab_t = jax.lax.dot_general(
            tab_ref[...], eye,
            dimension_numbers=(((0,), (0,)), ((), ())),
            preferred_element_type=jnp.float32,
            precision=jax.lax.Precision.HIGHEST)
        t0 = tab_t[:, 0:1]
        t256 = tab_t[:, 256:257]
        for s in range(_NSEG):
            seg_ref[s] = seg_cols((t0, tab_t, t256), s * _SEG, s * _SEG + _SEGW)

    return pl.pallas_call(
        body,
        out_shape=jax.ShapeDtypeStruct((_NSEG, _D, _SEGW), jnp.float32),
    )(table)


def _build_planes(segs):
    def body(seg_ref, pl_ref):
        p = pl.program_id(0)
        shifted = pltpu.roll(seg_ref[0], -p, 1)
        pl_ref[0] = shifted[:, :_SEG]

    return pl.pallas_call(
        body,
        grid=(_NPH, _NSEG),
        in_specs=[pl.BlockSpec((1, _D, _SEGW), lambda p, s: (s, 0, 0))],
        out_specs=pl.BlockSpec((1, _D, _SEG), lambda p, s: (p, 0, s)),
        out_shape=jax.ShapeDtypeStruct((_NPH, _D, _PCOLS), jnp.float32),
    )(segs)


_SC_EXPAND_CACHE = []


def _get_sc_expand():
    if _SC_EXPAND_CACHE:
        return _SC_EXPAND_CACHE[0]
    f = functools.partial(
        pl.kernel,
        mesh=plsc.VectorSubcoreMesh(core_axis_name="c", subcore_axis_name="s"),
        out_type=jax.ShapeDtypeStruct((_L, _D, _L), jnp.float32),
        scratch_types=[
            pltpu.VMEM((_D, _CHUNK), jnp.float32),
            pltpu.VMEM((_D, _CHUNK), jnp.float32),
            pltpu.VMEM((16,), jnp.int32),
            pltpu.SemaphoreType.DMA,
            pltpu.SemaphoreType.DMA,
        ],
    )(_sc_expand_body)
    _SC_EXPAND_CACHE.append(f)
    return f


def _sc_expand_body(p_hbm, delta_hbm, out_hbm, buf_a, buf_b, delta_v, sem_st, sem_em):
    cid = lax.axis_index("c")
    sid = lax.axis_index("s")

    pltpu.sync_copy(delta_hbm, delta_v)
    delta = delta_v[...][0]

    wid = cid * _NS + sid
    base = wid * _ROWS_PER_W

    def slab(s):
        c = lax.div(s, _ROWS_PER_W)
        r = lax.rem(s, _ROWS_PER_W)
        i = base + r
        start = jnp.clip(delta - i, _SMIN, _SMAX) - _SMIN
        p = jnp.bitwise_and(start, 127)
        al = pl.multiple_of(
            jnp.bitwise_and(start, -128) + c * _CHUNK, 128)
        cc = pl.multiple_of(c * _CHUNK, 128)
        return p, al, i, cc

    def stage(s, buf):
        p, al, i, cc = slab(s)
        pltpu.make_async_copy(
            p_hbm.at[p, :, pl.ds(al, _CHUNK)], buf, sem_st).start()

    def emit(s, buf):
        p, al, i, cc = slab(s)
        pltpu.make_async_copy(
            buf, out_hbm.at[i, :, pl.ds(cc, _CHUNK)], sem_em).start()

    def wait_st():
        pltpu.make_async_copy(
            p_hbm.at[0, :, pl.ds(0, _CHUNK)], buf_a, sem_st).wait()

    def wait_em():
        pltpu.make_async_copy(
            buf_a, out_hbm.at[base, :, pl.ds(0, _CHUNK)], sem_em).wait()

    stage(0, buf_a)
    wait_st()

    def body(s2, carry):
        s = 2 * s2
        stage(s + 1, buf_b)
        emit(s, buf_a)
        wait_em()
        wait_st()

        @pl.when(s2 <= _NSLAB // 2 - 2)
        def _():
            stage(s + 2, buf_a)

        emit(s + 1, buf_b)
        wait_em()

        @pl.when(s2 <= _NSLAB // 2 - 2)
        def _():
            wait_st()

        return carry

    lax.fori_loop(0, _NSLAB // 2, body, 0)


def kernel(length_q, length_k, embeddings_table):
    delta = (jnp.asarray(length_k, jnp.int32)
             - jnp.asarray(length_q, jnp.int32))
    delta16 = jnp.broadcast_to(delta.reshape(1), (16,))
    segs = _build_gt_segs(embeddings_table)
    planes = _build_planes(segs)
    out_t = _get_sc_expand()(planes, delta16)
    return jnp.transpose(out_t, (0, 2, 1))


# speedup vs baseline: 2.9153x; 2.1923x over previous
import functools

import jax
import jax.numpy as jnp
from jax import lax
from jax.experimental import pallas as pl
from jax.experimental.pallas import tpu as pltpu
from jax.experimental.pallas import tpu_sc as plsc

_L = 2048
_V = 257
_D = 64
_GPAD = 4608
_SMIN = -(_L + 127)
_SMAX = 128
_SEG = 384
_SEGW = 512
_NSEG = 12
_NPH = 128
_BLO = 1536
_BHI = 2176
_BCOLS = 3 * _SEG

_NC = 2
_NS = 16
_NW = _NC * _NS
_ROWS_PER_W = _L // _NW
_CHUNK = 512
_NCHUNK = _L // _CHUNK
_NSLAB = _ROWS_PER_W * _NCHUNK
_INFLIGHT = 24


def _build_gt_segs(table):
    def seg_cols(gt_parts, a, b):
        t0, tab_t, t256 = gt_parts
        parts = []
        if a < 2048:
            parts.append(jnp.broadcast_to(t0, (_D, min(b, 2048) - a)))
        if b > 2048 and a < 2304:
            lo, hi = max(a, 2048), min(b, 2304)
            parts.append(tab_t[:, lo - 2047:hi - 2047])
        if b > 2304:
            lo = max(a, 2304)
            parts.append(jnp.broadcast_to(t256, (_D, b - lo)))
        return jnp.concatenate(parts, axis=1) if len(parts) > 1 else parts[0]

    def body(tab_ref, seg_ref, const_ref):
        eye = jnp.eye(_V, dtype=jnp.float32)
        tab_t = jax.lax.dot_general(
            tab_ref[...], eye,
            dimension_numbers=(((0,), (0,)), ((), ())),
            preferred_element_type=jnp.float32,
            precision=jax.lax.Precision.HIGHEST)
        t0 = tab_t[:, 0:1]
        t256 = tab_t[:, 256:257]
        for s in range(_NSEG):
            seg_ref[s] = seg_cols((t0, tab_t, t256), s * _SEG, s * _SEG + _SEGW)
        const_ref[0] = jnp.broadcast_to(t0, (_D, _CHUNK))
        const_ref[1] = jnp.broadcast_to(t256, (_D, _CHUNK))

    return pl.pallas_call(
        body,
        out_shape=(jax.ShapeDtypeStruct((_NSEG, _D, _SEGW), jnp.float32),
                   jax.ShapeDtypeStruct((2, _D, _CHUNK), jnp.float32)),
    )(table)


def _build_band_planes(band_segs):
    def body(seg_ref, pl_ref):
        p = pl.program_id(0)
        for s in range(3):
            shifted = pltpu.roll(seg_ref[s], -p, 1)
            pl_ref[0, :, s * _SEG:(s + 1) * _SEG] = shifted[:, :_SEG]

    return pl.pallas_call(
        body,
        grid=(_NPH,),
        in_specs=[pl.BlockSpec((3, _D, _SEGW), lambda p: (0, 0, 0))],
        out_specs=pl.BlockSpec((1, _D, _BCOLS), lambda p: (p, 0, 0)),
        out_shape=jax.ShapeDtypeStruct((_NPH, _D, _BCOLS), jnp.float32),
    )(band_segs)


_SC_EXPAND_CACHE = []


def _get_sc_expand():
    if _SC_EXPAND_CACHE:
        return _SC_EXPAND_CACHE[0]
    f = functools.partial(
        pl.kernel,
        mesh=plsc.VectorSubcoreMesh(core_axis_name="c", subcore_axis_name="s"),
        out_type=jax.ShapeDtypeStruct((_L, _D, _L), jnp.float32),
        scratch_types=[
            pltpu.VMEM((_D, _CHUNK), jnp.float32),
            pltpu.VMEM((_D, _CHUNK), jnp.float32),
            pltpu.VMEM((_D, _CHUNK), jnp.float32),
            pltpu.VMEM((16,), jnp.int32),
            pltpu.SemaphoreType.DMA,
            pltpu.SemaphoreType.DMA,
        ],
    )(_sc_expand_body)
    _SC_EXPAND_CACHE.append(f)
    return f


def _sc_expand_body(band_hbm, const_hbm, delta_hbm, out_hbm,
                    t0_buf, t256_buf, band_buf, delta_v, sem_st, sem_em):
    cid = lax.axis_index("c")
    sid = lax.axis_index("s")

    pltpu.sync_copy(delta_hbm, delta_v)
    delta = delta_v[...][0]

    pltpu.make_async_copy(const_hbm.at[0], t0_buf, sem_st).start()
    pltpu.make_async_copy(const_hbm.at[1], t256_buf, sem_st).start()
    pltpu.make_async_copy(const_hbm.at[0], t0_buf, sem_st).wait()
    pltpu.make_async_copy(const_hbm.at[1], t256_buf, sem_st).wait()

    wid = cid * _NS + sid
    base = wid * _ROWS_PER_W

    def emit_async(buf, i, cc):
        pltpu.make_async_copy(
            buf, out_hbm.at[i, :, pl.ds(cc, _CHUNK)], sem_em).start()

    def drain_one():
        pltpu.make_async_copy(
            t0_buf, out_hbm.at[base, :, pl.ds(0, _CHUNK)], sem_em).wait()

    def body(s, n_out):
        c = lax.div(s, _ROWS_PER_W)
        r = lax.rem(s, _ROWS_PER_W)
        i = base + r
        start = jnp.clip(delta - i, _SMIN, _SMAX) - _SMIN
        p = jnp.bitwise_and(start, 127)
        al = jnp.bitwise_and(start, -128) + c * _CHUNK
        cc = pl.multiple_of(c * _CHUNK, 128)

        is_t0 = al <= _BLO - _CHUNK + 384
        is_t256 = al >= _BHI + 128
        is_band = jnp.logical_not(jnp.logical_or(is_t0, is_t256))

        @pl.when(is_t0)
        def _():
            emit_async(t0_buf, i, cc)

        @pl.when(is_t256)
        def _():
            emit_async(t256_buf, i, cc)

        @pl.when(is_band)
        def _():
            bo = pl.multiple_of(al - _BLO, 128)
            pltpu.sync_copy(band_hbm.at[p, :, pl.ds(bo, _CHUNK)], band_buf)
            pltpu.sync_copy(band_buf, out_hbm.at[i, :, pl.ds(cc, _CHUNK)])

        n_out = n_out + jnp.where(is_band, 0, 1)

        @pl.when(n_out >= _INFLIGHT)
        def _():
            drain_one()

        return n_out - jnp.where(n_out >= _INFLIGHT, 1, 0)

    n_out = lax.fori_loop(0, _NSLAB, body, jnp.int32(0))

    def final_drain(k, carry):
        drain_one()
        return carry

    lax.fori_loop(0, n_out, final_drain, 0)


def kernel(length_q, length_k, embeddings_table):
    delta = (jnp.asarray(length_k, jnp.int32)
             - jnp.asarray(length_q, jnp.int32))
    delta16 = jnp.broadcast_to(delta.reshape(1), (16,))
    segs, const_slabs = _build_gt_segs(embeddings_table)
    band = _build_band_planes(segs[4:7])
    out_t = _get_sc_expand()(band, const_slabs, delta16)
    return jnp.transpose(out_t, (0, 2, 1))
